# baseline (speedup 1.0000x reference)
import jax
import jax.numpy as jnp
from jax.experimental import pallas as pl
from jax.experimental.pallas import tpu as pltpu

_N = 10000
_BM = 400
_BW = 200
_NS = _N // _BM
_W = 2


def _fused_kernel(x_ref, w1_ref, b1_ref, w2_ref, b2_ref, lw_ref, l_ref,
                  o_ref, s1_ref, s2_ref):
    i = pl.program_id(0)

    @pl.when(i == 0)
    def _():
        s1_ref[...] = jnp.dot(x_ref[...], w1_ref[...],
                              preferred_element_type=jnp.float32)

    def pass1(l, base, rows):
        h = jnp.dot(l, s1_ref[...], preferred_element_type=jnp.float32)
        h = jnp.maximum(h + b1_ref[...], 0.0)
        s2_ref[pl.ds(base, rows), :] = jnp.dot(
            h, w2_ref[...], preferred_element_type=jnp.float32)

    @pl.when(i < _W)
    def _():
        pass1(lw_ref[...], i * _BW, _BW)

    @pl.when((i >= _W) & (i <= _NS))
    def _():
        pass1(l_ref[...], (i - 1) * _BM, _BM)

    @pl.when(i > _NS)
    def _():
        o_ref[...] = jnp.dot(l_ref[...], s2_ref[...],
                             preferred_element_type=jnp.float32) + b2_ref[...]


def _lw_map(i):
    return (jnp.minimum(i, _W - 1), 0)


def _l_map(i):
    return (jnp.where(i < _W, 1,
                      jnp.where(i <= _NS, i - 1, 2 * _NS - i)), 0)


def _out_map(i):
    return (jnp.where(i > _NS, 2 * _NS - i, 0), 0)


def kernel(Laplacian, feature, W1, b1, W2, b2):
    n, in_dim = feature.shape
    n_hid = W1.shape[1]
    out_dim = W2.shape[1]
    b1r = b1.reshape(1, n_hid)
    b2r = b2.reshape(1, out_dim)

    return pl.pallas_call(
        _fused_kernel,
        grid=(2 * _NS + 1,),
        in_specs=[
            pl.BlockSpec((n, in_dim), lambda i: (0, 0)),
            pl.BlockSpec((in_dim, n_hid), lambda i: (0, 0)),
            pl.BlockSpec((1, n_hid), lambda i: (0, 0)),
            pl.BlockSpec((n_hid, out_dim), lambda i: (0, 0)),
            pl.BlockSpec((1, out_dim), lambda i: (0, 0)),
            pl.BlockSpec((_BW, n), _lw_map),
            pl.BlockSpec((_BM, n), _l_map),
        ],
        out_specs=pl.BlockSpec((_BM, out_dim), _out_map),
        out_shape=jax.ShapeDtypeStruct((n, out_dim), jnp.float32),
        scratch_shapes=[
            pltpu.VMEM((n, n_hid), jnp.float32),
            pltpu.VMEM((n, out_dim), jnp.float32),
        ],
        compiler_params=pltpu.CompilerParams(
            dimension_semantics=("arbitrary",),
            vmem_limit_bytes=67108864),
    )(feature, W1, b1r, W2, b2r, Laplacian, Laplacian)


# speedup vs baseline: 1.0163x; 1.0163x over previous
import jax
import jax.numpy as jnp
from jax.experimental import pallas as pl
from jax.experimental.pallas import tpu as pltpu

_N = 10000
_BM = 400
_NS = _N // _BM


def _fused_kernel(x_ref, w1_ref, b1_ref, w2_ref, b2_ref, l_ref,
                  o_ref, s1_ref, s2_ref):
    i = pl.program_id(0)

    @pl.when(i == 0)
    def _():
        s1_ref[...] = jnp.dot(x_ref[...], w1_ref[...],
                              preferred_element_type=jnp.float32)

    @pl.when(i < _NS)
    def _():
        h = jnp.dot(l_ref[...], s1_ref[...],
                    preferred_element_type=jnp.float32)
        h = jnp.maximum(h + b1_ref[...], 0.0)
        s2_ref[pl.ds(i * _BM, _BM), :] = jnp.dot(
            h, w2_ref[...], preferred_element_type=jnp.float32)

    @pl.when(i >= _NS)
    def _():
        o_ref[...] = jnp.dot(l_ref[...], s2_ref[...],
                             preferred_element_type=jnp.float32) + b2_ref[...]


def _l_stripe(i):
    return (jnp.where(i < _NS, i, 2 * _NS - 1 - i), 0)


def _out_stripe(i):
    return (jnp.where(i >= _NS, 2 * _NS - 1 - i, 0), 0)


def kernel(Laplacian, feature, W1, b1, W2, b2):
    n, in_dim = feature.shape
    n_hid = W1.shape[1]
    out_dim = W2.shape[1]
    b1r = b1.reshape(1, n_hid)
    b2r = b2.reshape(1, out_dim)

    return pl.pallas_call(
        _fused_kernel,
        grid=(2 * _NS,),
        in_specs=[
            pl.BlockSpec((n, in_dim), lambda i: (0, 0)),
            pl.BlockSpec((in_dim, n_hid), lambda i: (0, 0)),
            pl.BlockSpec((1, n_hid), lambda i: (0, 0)),
            pl.BlockSpec((n_hid, out_dim), lambda i: (0, 0)),
            pl.BlockSpec((1, out_dim), lambda i: (0, 0)),
            pl.BlockSpec((_BM, n), _l_stripe),
        ],
        out_specs=pl.BlockSpec((_BM, out_dim), _out_stripe),
        out_shape=jax.ShapeDtypeStruct((n, out_dim), jnp.float32),
        scratch_shapes=[
            pltpu.VMEM((n, n_hid), jnp.float32),
            pltpu.VMEM((n, out_dim), jnp.float32),
        ],
        compiler_params=pltpu.CompilerParams(
            dimension_semantics=("arbitrary",)),
    )(feature, W1, b1r, W2, b2r, Laplacian)
